# baseline (speedup 1.0000x reference)
import functools

import jax
import jax.numpy as jnp
from jax import lax
from jax.experimental import pallas as pl
from jax.experimental.pallas import tpu as pltpu
from jax.experimental.pallas import tpu_sc as plsc

_SPREAD = 1.0e-4
_LAMBDA_TC = 0.5
_LAMBDA_CVAR = 0.1
_TARGET_VOL = 0.001
_QUANTILE = 0.1
_LAMBDA_L2 = 0.02

_N = 16384
_M = 200
_TOT = _N * _M
_K = max(1, int(_QUANTILE * _TOT))

_NW = 32
_RPW = _N // _NW
_EPW = _RPW * _M
_B = 4096

_F32MAX = 3.4e38


def _rsqrt_sc(x):
    i = lax.bitcast_convert_type(x, jnp.int32)
    i = jnp.int32(0x5F3759DF) - lax.shift_right_logical(i, 1)
    y = lax.bitcast_convert_type(i, jnp.float32)
    for _ in range(3):
        y = y * (1.5 - 0.5 * x * y * y)
    return y


def _sc_body(tflat_hbm, sig_hbm, prev_hbm, cnt_hbm, sum_hbm, st_hbm,
             data_v, sig_v, prev_v, pos_v, cnt_v, sum_v, st_v,
             semA, semB):
    wid = lax.axis_index("c") * 16 + lax.axis_index("s")

    _H = _EPW // 2
    cpA = pltpu.async_copy(tflat_hbm.at[pl.ds(wid * _EPW, _H)],
                           data_v.at[pl.ds(0, _H)], semA)
    cpB = pltpu.async_copy(tflat_hbm.at[pl.ds(wid * _EPW + _H, _H)],
                           data_v.at[pl.ds(_H, _H)], semB)
    pltpu.sync_copy(sig_hbm.at[pl.ds(wid * _RPW, _RPW)],
                    sig_v.at[pl.ds(0, _RPW)])
    pltpu.sync_copy(prev_hbm.at[pl.ds(wid * _RPW, _RPW)],
                    prev_v.at[pl.ds(0, _RPW)])

    lane = lax.iota(jnp.int32, 16)
    lane_lt8 = lane < 8
    zz = jnp.zeros((16,), jnp.float32)

    @plsc.parallel_loop(0, _B // 16, step=1, unroll=8)
    def _zero(i):
        cnt_v[pl.ds(i * 16, 16)] = zz
        sum_v[pl.ds(i * 16, 16)] = zz

    init = (zz, jnp.full((16,), _F32MAX, jnp.float32),
            jnp.full((16,), -_F32MAX, jnp.float32))
    cpA.wait()

    def _pair_body(r, c):
        acc_pnl, acc_mn, acc_mx = c
        base = r * (2 * _M)
        v = [data_v[pl.ds(base + k * 16, 16)] for k in range(25)]
        half0 = lambda x, fill: jnp.where(lane_lt8, x, fill)
        half1 = lambda x, fill: jnp.where(lane_lt8, fill, x)

        s0v = v[0]
        q0v = v[0] * v[0]
        mn0v = v[0]
        mx0v = v[0]
        for k in range(1, 12):
            s0v = s0v + v[k]
            q0v = q0v + v[k] * v[k]
            mn0v = jnp.minimum(mn0v, v[k])
            mx0v = jnp.maximum(mx0v, v[k])
        s0v = s0v + half0(v[12], 0.0)
        q0v = q0v + half0(v[12] * v[12], 0.0)
        mn0v = jnp.minimum(mn0v, half0(v[12], _F32MAX))
        mx0v = jnp.maximum(mx0v, half0(v[12], -_F32MAX))

        s1v = v[13]
        q1v = v[13] * v[13]
        mn1v = v[13]
        mx1v = v[13]
        for k in range(14, 25):
            s1v = s1v + v[k]
            q1v = q1v + v[k] * v[k]
            mn1v = jnp.minimum(mn1v, v[k])
            mx1v = jnp.maximum(mx1v, v[k])
        s1v = s1v + half1(v[12], 0.0)
        q1v = q1v + half1(v[12] * v[12], 0.0)
        mn1v = jnp.minimum(mn1v, half1(v[12], _F32MAX))
        mx1v = jnp.maximum(mx1v, half1(v[12], -_F32MAX))

        sgp = sig_v[pl.ds(2 * r, 16)]
        for (sv, qv, mnv, mxv, li) in ((s0v, q0v, mn0v, mx0v, 0),
                                       (s1v, q1v, mn1v, mx1v, 1)):
            s = jnp.full((16,), jnp.sum(sv), jnp.float32)
            q = jnp.full((16,), jnp.sum(qv), jnp.float32)
            var = (q - s * s * (1.0 / _M)) * (1.0 / (_M - 1))
            var = jnp.maximum(var, 1e-16)
            y = _rsqrt_sc(var)
            vol = jnp.maximum(var * y, 1e-8)
            scale = jnp.clip(_TARGET_VOL / vol, 0.1, 3.0)
            sg = jnp.full((16,), sgp[li], jnp.float32)
            pos = sg * scale
            pos_v[pl.ds((2 * r + li) * 16, 16)] = pos
            acc_pnl = acc_pnl + pos * s
            acc_mn = jnp.minimum(acc_mn,
                                 jnp.minimum(pos * mnv, pos * mxv))
            acc_mx = jnp.maximum(acc_mx,
                                 jnp.maximum(pos * mnv, pos * mxv))
        return (acc_pnl, acc_mn, acc_mx)

    mid = plsc.parallel_loop(0, _RPW // 4, step=1, unroll=4,
                             carry=init)(_pair_body)
    cpB.wait()
    acc_pnl, acc_mn, acc_mx = plsc.parallel_loop(
        _RPW // 4, _RPW // 2, step=1, unroll=4, carry=mid)(_pair_body)

    init2 = (zz, zz)

    @plsc.parallel_loop(0, _RPW // 16, step=1, unroll=4, carry=init2)
    def _sig(i, c):
        a, l = c
        sv = sig_v[pl.ds(i * 16, 16)]
        pv = prev_v[pl.ds(i * 16, 16)]
        return (a + jnp.abs(sv - pv), l + sv * sv)

    acc_abs, acc_l2 = _sig

    mn_t = jnp.full((16,), jnp.min(acc_mn), jnp.float32)
    mx_t = jnp.full((16,), jnp.max(acc_mx), jnp.float32)
    invw = float(_B) / jnp.maximum(mx_t - mn_t, 1e-30)
    onev = jnp.full((16,), 1.0, jnp.float32)
    bmax = jnp.full((16,), _B - 1, jnp.int32)
    bmin = jnp.full((16,), 0, jnp.int32)

    @plsc.parallel_loop(0, _RPW // 2, step=1, unroll=2)
    def _hist(r):
        p0 = pos_v[pl.ds((2 * r) * 16, 16)]
        p1 = pos_v[pl.ds((2 * r + 1) * 16, 16)]
        pm = jnp.where(lane_lt8, p0, p1)
        base = r * (2 * _M)
        for k in range(25):
            pv = p0 if k < 12 else (pm if k == 12 else p1)
            x = data_v[pl.ds(base + k * 16, 16)]
            bar = pv * x
            b = ((bar - mn_t) * invw).astype(jnp.int32)
            b = jnp.minimum(jnp.maximum(b, bmin), bmax)
            plsc.addupdate_scatter(cnt_v, [b], onev)
            plsc.addupdate_scatter(sum_v, [b], bar)

    st_v[pl.ds(0, 16)] = acc_pnl
    st_v[pl.ds(16, 16)] = acc_abs
    st_v[pl.ds(32, 16)] = acc_l2
    st_v[pl.ds(48, 16)] = mn_t
    st_v[pl.ds(64, 16)] = mx_t
    st_v[pl.ds(80, 16)] = zz
    st_v[pl.ds(96, 16)] = zz
    st_v[pl.ds(112, 16)] = zz

    pltpu.sync_copy(cnt_v, cnt_hbm.at[wid])
    pltpu.sync_copy(sum_v, sum_hbm.at[wid])
    pltpu.sync_copy(st_v, st_hbm.at[wid])


_sc_call = functools.partial(
    pl.kernel,
    mesh=plsc.VectorSubcoreMesh(core_axis_name="c", subcore_axis_name="s"),
    out_type=[
        jax.ShapeDtypeStruct((_NW, _B), jnp.float32),
        jax.ShapeDtypeStruct((_NW, _B), jnp.float32),
        jax.ShapeDtypeStruct((_NW, 128), jnp.float32),
    ],
    scratch_types=[
        pltpu.VMEM((_EPW,), jnp.float32),
        pltpu.VMEM((_RPW + 16,), jnp.float32),
        pltpu.VMEM((_RPW + 16,), jnp.float32),
        pltpu.VMEM((_RPW * 16,), jnp.float32),
        pltpu.VMEM((_B,), jnp.float32),
        pltpu.VMEM((_B,), jnp.float32),
        pltpu.VMEM((128,), jnp.float32),
        pltpu.SemaphoreType.DMA,
        pltpu.SemaphoreType.DMA,
    ],
    compiler_params=pltpu.CompilerParams(needs_layout_passes=False),
)(_sc_body)


def _merge_body(cnt_ref, sum_ref, st_ref, out_ref):
    cnt = cnt_ref[...]
    sm = sum_ref[...]
    st = st_ref[...]
    lane = lax.broadcasted_iota(jnp.int32, (_NW, 128), 1)
    col0 = (lane == 0).astype(jnp.float32)
    pnl_sum = jnp.sum(jnp.where(lane == 0, st, 0.0))
    abs_sum = jnp.sum(jnp.where((lane >= 16) & (lane < 32), st, 0.0))
    sq_sum = jnp.sum(jnp.where((lane >= 32) & (lane < 48), st, 0.0))
    mn_w = jnp.sum(jnp.where(lane == 48, st, 0.0), axis=1, keepdims=True)
    mx_w = jnp.sum(jnp.where(lane == 64, st, 0.0), axis=1, keepdims=True)
    w_w = (mx_w - mn_w) * (1.0 / _B)

    j = lax.broadcasted_iota(jnp.int32, (_NW, _B), 1).astype(jnp.float32)
    upper = mn_w + (j + 1.0) * w_w
    lower = mn_w + j * w_w

    kf = jnp.float32(_K)

    lo = jnp.min(mn_w)
    hi = jnp.max(mx_w) + 1e-30

    def level(tl, th):
        step = (th - tl) * (1.0 / 16.0)
        best = tl
        for i in range(1, 17):
            t = tl + step * jnp.float32(i)
            cf = jnp.sum(jnp.where(upper <= t, cnt, 0.0))
            best = jnp.where(cf <= kf, t, best)
        return best, best + step

    t1, t1h = level(lo, hi)
    t2, _ = level(t1, t1h)

    fully = (upper <= t2).astype(jnp.float32)
    strad = jnp.logical_and(lower < t2, upper > t2).astype(jnp.float32)
    c_b = jnp.sum(cnt * fully)
    s_b = jnp.sum(sm * fully)
    sc_cnt = jnp.sum(cnt * strad)
    sc_sum = jnp.sum(sm * strad)
    t_hat = sc_sum / jnp.maximum(sc_cnt, 1.0)
    s_k = s_b + (kf - c_b) * t_hat

    turnover = abs_sum * (1.0 / _N)
    tc_cost = _LAMBDA_TC * turnover * _SPREAD
    cvar = -(s_k * (1.0 / _K))
    loss = (-(pnl_sum * (1.0 / _N)) + tc_cost + _LAMBDA_CVAR * cvar
            + _LAMBDA_L2 * (sq_sum * (1.0 / _N)))
    del col0
    out_ref[...] = jnp.full((1, 1), loss, jnp.float32)


_merge_call = pl.pallas_call(
    _merge_body,
    in_specs=[
        pl.BlockSpec((_NW, _B), lambda: (0, 0)),
        pl.BlockSpec((_NW, _B), lambda: (0, 0)),
        pl.BlockSpec((_NW, 128), lambda: (0, 0)),
    ],
    out_specs=pl.BlockSpec((1, 1), lambda: (0, 0)),
    out_shape=jax.ShapeDtypeStruct((1, 1), jnp.float32),
)


def kernel(signal, targets, prev_sig):
    cnt, sm, st = _sc_call(targets.reshape(-1), signal.reshape(-1), prev_sig)
    out = _merge_call(cnt, sm, st)
    return out[0, 0]


# speedup vs baseline: 1.1727x; 1.1727x over previous
import functools

import jax
import jax.numpy as jnp
from jax import lax
from jax.experimental import pallas as pl
from jax.experimental.pallas import tpu as pltpu
from jax.experimental.pallas import tpu_sc as plsc

_SPREAD = 1.0e-4
_LAMBDA_TC = 0.5
_LAMBDA_CVAR = 0.1
_TARGET_VOL = 0.001
_QUANTILE = 0.1
_LAMBDA_L2 = 0.02

_N = 16384
_M = 200
_TOT = _N * _M
_K = max(1, int(_QUANTILE * _TOT))

_NW = 32
_RPW = _N // _NW
_EPW = _RPW * _M
_B = 4096

_F32MAX = 3.4e38


def _rsqrt_sc(x):
    i = lax.bitcast_convert_type(x, jnp.int32)
    i = jnp.int32(0x5F3759DF) - lax.shift_right_logical(i, 1)
    y = lax.bitcast_convert_type(i, jnp.float32)
    for _ in range(3):
        y = y * (1.5 - 0.5 * x * y * y)
    return y


def _sc_body(tflat_hbm, sig_hbm, prev_hbm, cnt_hbm, sum_hbm, st_hbm,
             data_v, sig_v, prev_v, pos_v, cnt_v, sum_v, st_v,
             semA, semB):
    wid = lax.axis_index("c") * 16 + lax.axis_index("s")

    _H = _EPW // 2
    cpA = pltpu.async_copy(tflat_hbm.at[pl.ds(wid * _EPW, _H)],
                           data_v.at[pl.ds(0, _H)], semA)
    cpB = pltpu.async_copy(tflat_hbm.at[pl.ds(wid * _EPW + _H, _H)],
                           data_v.at[pl.ds(_H, _H)], semB)
    pltpu.sync_copy(sig_hbm.at[pl.ds(wid * _RPW, _RPW)],
                    sig_v.at[pl.ds(0, _RPW)])
    pltpu.sync_copy(prev_hbm.at[pl.ds(wid * _RPW, _RPW)],
                    prev_v.at[pl.ds(0, _RPW)])

    lane = lax.iota(jnp.int32, 16)
    lane_lt8 = lane < 8
    zz = jnp.zeros((16,), jnp.float32)

    @plsc.parallel_loop(0, _B // 16, step=1, unroll=8)
    def _zero(i):
        cnt_v[pl.ds(i * 16, 16)] = zz
        sum_v[pl.ds(i * 16, 16)] = zz

    init = (zz, jnp.full((16,), _F32MAX, jnp.float32),
            jnp.full((16,), -_F32MAX, jnp.float32))
    cpA.wait()

    def _pair_body(r, c):
        acc_pnl, acc_mn, acc_mx = c
        base = r * (2 * _M)
        v = [data_v[pl.ds(base + k * 16, 16)] for k in range(25)]
        half0 = lambda x, fill: jnp.where(lane_lt8, x, fill)
        half1 = lambda x, fill: jnp.where(lane_lt8, fill, x)

        s0v = v[0]
        q0v = v[0] * v[0]
        mn0v = v[0]
        mx0v = v[0]
        for k in range(1, 12):
            s0v = s0v + v[k]
            q0v = q0v + v[k] * v[k]
            mn0v = jnp.minimum(mn0v, v[k])
            mx0v = jnp.maximum(mx0v, v[k])
        s0v = s0v + half0(v[12], 0.0)
        q0v = q0v + half0(v[12] * v[12], 0.0)
        mn0v = jnp.minimum(mn0v, half0(v[12], _F32MAX))
        mx0v = jnp.maximum(mx0v, half0(v[12], -_F32MAX))

        s1v = v[13]
        q1v = v[13] * v[13]
        mn1v = v[13]
        mx1v = v[13]
        for k in range(14, 25):
            s1v = s1v + v[k]
            q1v = q1v + v[k] * v[k]
            mn1v = jnp.minimum(mn1v, v[k])
            mx1v = jnp.maximum(mx1v, v[k])
        s1v = s1v + half1(v[12], 0.0)
        q1v = q1v + half1(v[12] * v[12], 0.0)
        mn1v = jnp.minimum(mn1v, half1(v[12], _F32MAX))
        mx1v = jnp.maximum(mx1v, half1(v[12], -_F32MAX))

        sgp = sig_v[pl.ds(2 * r, 16)]
        for (sv, qv, mnv, mxv, li) in ((s0v, q0v, mn0v, mx0v, 0),
                                       (s1v, q1v, mn1v, mx1v, 1)):
            s = jnp.full((16,), jnp.sum(sv), jnp.float32)
            q = jnp.full((16,), jnp.sum(qv), jnp.float32)
            var = (q - s * s * (1.0 / _M)) * (1.0 / (_M - 1))
            var = jnp.maximum(var, 1e-16)
            y = _rsqrt_sc(var)
            vol = jnp.maximum(var * y, 1e-8)
            scale = jnp.clip(_TARGET_VOL / vol, 0.1, 3.0)
            sg = jnp.full((16,), sgp[li], jnp.float32)
            pos = sg * scale
            pos_v[pl.ds((2 * r + li) * 16, 16)] = pos
            acc_pnl = acc_pnl + pos * s
            acc_mn = jnp.minimum(acc_mn,
                                 jnp.minimum(pos * mnv, pos * mxv))
            acc_mx = jnp.maximum(acc_mx,
                                 jnp.maximum(pos * mnv, pos * mxv))
        return (acc_pnl, acc_mn, acc_mx)

    mid = plsc.parallel_loop(0, _RPW // 4, step=1, unroll=2,
                             carry=init)(_pair_body)
    cpB.wait()
    acc_pnl, acc_mn, acc_mx = plsc.parallel_loop(
        _RPW // 4, _RPW // 2, step=1, unroll=2, carry=mid)(_pair_body)

    init2 = (zz, zz)

    @plsc.parallel_loop(0, _RPW // 16, step=1, unroll=4, carry=init2)
    def _sig(i, c):
        a, l = c
        sv = sig_v[pl.ds(i * 16, 16)]
        pv = prev_v[pl.ds(i * 16, 16)]
        return (a + jnp.abs(sv - pv), l + sv * sv)

    acc_abs, acc_l2 = _sig

    mn_t = jnp.full((16,), jnp.min(acc_mn), jnp.float32)
    mx_t = jnp.full((16,), jnp.max(acc_mx), jnp.float32)
    invw = float(_B) / jnp.maximum(mx_t - mn_t, 1e-30)
    onev = jnp.full((16,), 1.0, jnp.float32)
    bmax = jnp.full((16,), _B - 1, jnp.int32)
    bmin = jnp.full((16,), 0, jnp.int32)

    @plsc.parallel_loop(0, _RPW // 2, step=1, unroll=2)
    def _hist(r):
        p0 = pos_v[pl.ds((2 * r) * 16, 16)]
        p1 = pos_v[pl.ds((2 * r + 1) * 16, 16)]
        pm = jnp.where(lane_lt8, p0, p1)
        base = r * (2 * _M)
        for k in range(25):
            pv = p0 if k < 12 else (pm if k == 12 else p1)
            x = data_v[pl.ds(base + k * 16, 16)]
            bar = pv * x
            b = ((bar - mn_t) * invw).astype(jnp.int32)
            b = jnp.minimum(jnp.maximum(b, bmin), bmax)
            plsc.addupdate_scatter(cnt_v, [b], onev)
            plsc.addupdate_scatter(sum_v, [b], bar)

    st_v[pl.ds(0, 16)] = acc_pnl
    st_v[pl.ds(16, 16)] = acc_abs
    st_v[pl.ds(32, 16)] = acc_l2
    st_v[pl.ds(48, 16)] = mn_t
    st_v[pl.ds(64, 16)] = mx_t
    st_v[pl.ds(80, 16)] = zz
    st_v[pl.ds(96, 16)] = zz
    st_v[pl.ds(112, 16)] = zz

    pltpu.sync_copy(cnt_v, cnt_hbm.at[wid])
    pltpu.sync_copy(sum_v, sum_hbm.at[wid])
    pltpu.sync_copy(st_v, st_hbm.at[wid])


_sc_call = functools.partial(
    pl.kernel,
    mesh=plsc.VectorSubcoreMesh(core_axis_name="c", subcore_axis_name="s"),
    out_type=[
        jax.ShapeDtypeStruct((_NW, _B), jnp.float32),
        jax.ShapeDtypeStruct((_NW, _B), jnp.float32),
        jax.ShapeDtypeStruct((_NW, 128), jnp.float32),
    ],
    scratch_types=[
        pltpu.VMEM((_EPW,), jnp.float32),
        pltpu.VMEM((_RPW + 16,), jnp.float32),
        pltpu.VMEM((_RPW + 16,), jnp.float32),
        pltpu.VMEM((_RPW * 16,), jnp.float32),
        pltpu.VMEM((_B,), jnp.float32),
        pltpu.VMEM((_B,), jnp.float32),
        pltpu.VMEM((128,), jnp.float32),
        pltpu.SemaphoreType.DMA,
        pltpu.SemaphoreType.DMA,
    ],
    compiler_params=pltpu.CompilerParams(needs_layout_passes=False),
)(_sc_body)


def _merge_body(cnt_ref, sum_ref, st_ref, out_ref):
    cnt = cnt_ref[...]
    sm = sum_ref[...]
    st = st_ref[...]
    lane = lax.broadcasted_iota(jnp.int32, (_NW, 128), 1)
    col0 = (lane == 0).astype(jnp.float32)
    pnl_sum = jnp.sum(jnp.where(lane == 0, st, 0.0))
    abs_sum = jnp.sum(jnp.where((lane >= 16) & (lane < 32), st, 0.0))
    sq_sum = jnp.sum(jnp.where((lane >= 32) & (lane < 48), st, 0.0))
    mn_w = jnp.sum(jnp.where(lane == 48, st, 0.0), axis=1, keepdims=True)
    mx_w = jnp.sum(jnp.where(lane == 64, st, 0.0), axis=1, keepdims=True)
    w_w = (mx_w - mn_w) * (1.0 / _B)

    j = lax.broadcasted_iota(jnp.int32, (_NW, _B), 1).astype(jnp.float32)
    upper = mn_w + (j + 1.0) * w_w
    lower = mn_w + j * w_w

    kf = jnp.float32(_K)

    lo = jnp.min(mn_w)
    hi = jnp.max(mx_w) + 1e-30

    def level(tl, th):
        step = (th - tl) * (1.0 / 16.0)
        best = tl
        for i in range(1, 17):
            t = tl + step * jnp.float32(i)
            cf = jnp.sum(jnp.where(upper <= t, cnt, 0.0))
            best = jnp.where(cf <= kf, t, best)
        return best, best + step

    t1, t1h = level(lo, hi)
    t2, _ = level(t1, t1h)

    fully = (upper <= t2).astype(jnp.float32)
    strad = jnp.logical_and(lower < t2, upper > t2).astype(jnp.float32)
    c_b = jnp.sum(cnt * fully)
    s_b = jnp.sum(sm * fully)
    sc_cnt = jnp.sum(cnt * strad)
    sc_sum = jnp.sum(sm * strad)
    t_hat = sc_sum / jnp.maximum(sc_cnt, 1.0)
    s_k = s_b + (kf - c_b) * t_hat

    turnover = abs_sum * (1.0 / _N)
    tc_cost = _LAMBDA_TC * turnover * _SPREAD
    cvar = -(s_k * (1.0 / _K))
    loss = (-(pnl_sum * (1.0 / _N)) + tc_cost + _LAMBDA_CVAR * cvar
            + _LAMBDA_L2 * (sq_sum * (1.0 / _N)))
    del col0
    out_ref[...] = jnp.full((1, 1), loss, jnp.float32)


_merge_call = pl.pallas_call(
    _merge_body,
    in_specs=[
        pl.BlockSpec((_NW, _B), lambda: (0, 0)),
        pl.BlockSpec((_NW, _B), lambda: (0, 0)),
        pl.BlockSpec((_NW, 128), lambda: (0, 0)),
    ],
    out_specs=pl.BlockSpec((1, 1), lambda: (0, 0)),
    out_shape=jax.ShapeDtypeStruct((1, 1), jnp.float32),
)


def kernel(signal, targets, prev_sig):
    cnt, sm, st = _sc_call(targets.reshape(-1), signal.reshape(-1), prev_sig)
    out = _merge_call(cnt, sm, st)
    return out[0, 0]
